# baseline (speedup 1.0000x reference)
import functools

import jax
import jax.numpy as jnp
from jax import lax
from jax.experimental import pallas as pl
from jax.experimental.pallas import tpu as pltpu
from jax.experimental.pallas import tpu_sc as plsc

VOCAB_SIZE = 100000
EMB_DIM = 100
EMB_PAD = 128
BATCH_SIZE = 1024
VT = 2048
NV = (VOCAB_SIZE + VT - 1) // VT


def _padt_body(et_ref, out_ref):
    z = et_ref[...]
    z = jnp.concatenate(
        [z, jnp.zeros((EMB_PAD - EMB_DIM, VT), jnp.float32)], axis=0)
    out_ref[...] = z.T


def _prep_table(Et):
    return pl.pallas_call(
        _padt_body,
        grid=(NV,),
        in_specs=[pl.BlockSpec((EMB_DIM, VT), lambda j: (0, j))],
        out_specs=pl.BlockSpec((VT, EMB_PAD), lambda j: (j, 0)),
        out_shape=jax.ShapeDtypeStruct((VOCAB_SIZE, EMB_PAD), jnp.float32),
        compiler_params=pltpu.CompilerParams(
            dimension_semantics=("arbitrary",)),
    )(Et)


def _sc_gather(table, idx_flat):
    nc, ns = 2, 16
    nw = nc * ns
    n_idx = idx_flat.shape[0]
    per_w = n_idx // nw
    mesh = plsc.VectorSubcoreMesh(core_axis_name="c", subcore_axis_name="s",
                                  num_cores=nc, num_subcores=ns)

    @functools.partial(
        pl.kernel,
        mesh=mesh,
        out_type=jax.ShapeDtypeStruct((n_idx, EMB_PAD), jnp.float32),
        scratch_types=[
            pltpu.VMEM((per_w,), jnp.int32),
            pltpu.VMEM((per_w, EMB_PAD), jnp.float32),
            pltpu.SemaphoreType.DMA,
        ],
    )
    def gather_kernel(table_hbm, idx_hbm, out_hbm, idx_v, rows_v, sem):
        wid = lax.axis_index("s") * nc + lax.axis_index("c")
        base = wid * per_w
        pltpu.sync_copy(idx_hbm.at[pl.ds(base, per_w)], idx_v)
        pltpu.async_copy(table_hbm.at[idx_v], rows_v, sem).wait()
        pltpu.sync_copy(rows_v, out_hbm.at[pl.ds(base, per_w)])

    return gather_kernel(table, idx_flat)


def _build_embt(rows_t_ref, embt_ref):
    embt_ref[0:EMB_DIM, :] = rows_t_ref[
        0:EMB_DIM, 0:BATCH_SIZE].astype(jnp.bfloat16)
    embt_ref[EMB_DIM:2 * EMB_DIM, :] = rows_t_ref[
        0:EMB_DIM, BATCH_SIZE:2 * BATCH_SIZE].astype(jnp.bfloat16)


def _logits_t(w_ref, b_ref, embt_ref):
    wb = w_ref[...].astype(jnp.bfloat16)
    zt = lax.dot_general(wb, embt_ref[...],
                         (((0,), (0,)), ((), ())),
                         preferred_element_type=jnp.float32)
    bcol = lax.dot_general(b_ref[0], jnp.ones((1, 1), jnp.float32),
                           (((0,), (0,)), ((), ())),
                           preferred_element_type=jnp.float32)
    return zt + bcol


def _pass1_body(rows_t_ref, w_ref, b_ref, m_ref, l_ref, embt_ref):
    j = pl.program_id(0)

    @pl.when(j == 0)
    def _():
        _build_embt(rows_t_ref, embt_ref)
        m_ref[...] = jnp.zeros_like(m_ref)
        l_ref[...] = jnp.zeros_like(l_ref)

    r = jnp.maximum(_logits_t(w_ref, b_ref, embt_ref), 0.0)
    r = lax.cond(
        j == NV - 1,
        lambda rr: jnp.where(
            j * VT + lax.broadcasted_iota(jnp.int32, rr.shape, 0)
            < VOCAB_SIZE, rr, -jnp.inf),
        lambda rr: rr,
        r)
    tile_max = jnp.max(r, axis=0, keepdims=True)
    m_old = m_ref[...]
    m_new = jnp.maximum(m_old, tile_max)
    e = jnp.exp(r - m_new)
    l_ref[...] = l_ref[...] * jnp.exp(m_old - m_new) + jnp.sum(
        e, axis=0, keepdims=True)
    m_ref[...] = m_new


def _pass2_body(rows_t_ref, w_ref, b_ref, m_ref, l_ref, out_ref, embt_ref):
    j = pl.program_id(0)

    @pl.when(j == 0)
    def _():
        _build_embt(rows_t_ref, embt_ref)

    r = jnp.maximum(_logits_t(w_ref, b_ref, embt_ref), 0.0)
    out_ref[...] = jnp.exp(r - m_ref[...]) * (1.0 / l_ref[...])


def kernel(inputs, E, W, b):
    table = _prep_table(E.T)
    idx = jnp.concatenate([inputs[:, 0], inputs[:, 1]]).astype(jnp.int32)
    rows = _sc_gather(table, idx)
    rows_t = rows.T
    b3 = jnp.pad(b, (0, NV * VT - VOCAB_SIZE)).reshape(NV, 1, VT)

    m, l = pl.pallas_call(
        _pass1_body,
        grid=(NV,),
        in_specs=[
            pl.BlockSpec((EMB_PAD, 2 * BATCH_SIZE), lambda j: (0, 0)),
            pl.BlockSpec((2 * EMB_DIM, VT), lambda j: (0, j)),
            pl.BlockSpec((1, 1, VT), lambda j: (j, 0, 0)),
        ],
        out_specs=[
            pl.BlockSpec((1, BATCH_SIZE), lambda j: (0, 0)),
            pl.BlockSpec((1, BATCH_SIZE), lambda j: (0, 0)),
        ],
        out_shape=[
            jax.ShapeDtypeStruct((1, BATCH_SIZE), jnp.float32),
            jax.ShapeDtypeStruct((1, BATCH_SIZE), jnp.float32),
        ],
        scratch_shapes=[pltpu.VMEM((2 * EMB_DIM, BATCH_SIZE), jnp.bfloat16)],
        compiler_params=pltpu.CompilerParams(
            dimension_semantics=("arbitrary",)),
    )(rows_t, W, b3)

    out_t = pl.pallas_call(
        _pass2_body,
        grid=(NV,),
        in_specs=[
            pl.BlockSpec((EMB_PAD, 2 * BATCH_SIZE), lambda j: (0, 0)),
            pl.BlockSpec((2 * EMB_DIM, VT), lambda j: (0, j)),
            pl.BlockSpec((1, 1, VT), lambda j: (j, 0, 0)),
            pl.BlockSpec((1, BATCH_SIZE), lambda j: (0, 0)),
            pl.BlockSpec((1, BATCH_SIZE), lambda j: (0, 0)),
        ],
        out_specs=pl.BlockSpec((VT, BATCH_SIZE), lambda j: (j, 0)),
        out_shape=jax.ShapeDtypeStruct((VOCAB_SIZE, BATCH_SIZE), jnp.float32),
        scratch_shapes=[pltpu.VMEM((2 * EMB_DIM, BATCH_SIZE), jnp.bfloat16)],
        compiler_params=pltpu.CompilerParams(
            dimension_semantics=("arbitrary",)),
    )(rows_t, W, b3, m, l)
    return out_t.T


# speedup vs baseline: 1.2898x; 1.2898x over previous
import functools

import jax
import jax.numpy as jnp
from jax import lax
from jax.experimental import pallas as pl
from jax.experimental.pallas import tpu as pltpu
from jax.experimental.pallas import tpu_sc as plsc

VOCAB_SIZE = 100000
EMB_DIM = 100
EMB_PAD = 128
BATCH_SIZE = 1024
VT = 2048
NV = (VOCAB_SIZE + VT - 1) // VT


def _padt_body(et_ref, out_ref):
    z = et_ref[...]
    z = jnp.concatenate(
        [z, jnp.zeros((EMB_PAD - EMB_DIM, VT), jnp.float32)], axis=0)
    out_ref[...] = z.T


def _prep_table(Et):
    return pl.pallas_call(
        _padt_body,
        grid=(NV,),
        in_specs=[pl.BlockSpec((EMB_DIM, VT), lambda j: (0, j))],
        out_specs=pl.BlockSpec((VT, EMB_PAD), lambda j: (j, 0)),
        out_shape=jax.ShapeDtypeStruct((VOCAB_SIZE, EMB_PAD), jnp.float32),
        compiler_params=pltpu.CompilerParams(
            dimension_semantics=("arbitrary",)),
    )(Et)


def _sc_gather(table, idx_flat):
    nc, ns = 2, 16
    nw = nc * ns
    n_idx = idx_flat.shape[0]
    per_w = n_idx // nw
    mesh = plsc.VectorSubcoreMesh(core_axis_name="c", subcore_axis_name="s",
                                  num_cores=nc, num_subcores=ns)

    @functools.partial(
        pl.kernel,
        mesh=mesh,
        out_type=jax.ShapeDtypeStruct((n_idx, EMB_PAD), jnp.float32),
        scratch_types=[
            pltpu.VMEM((per_w,), jnp.int32),
            pltpu.VMEM((per_w, EMB_PAD), jnp.float32),
            pltpu.SemaphoreType.DMA,
        ],
    )
    def gather_kernel(table_hbm, idx_hbm, out_hbm, idx_v, rows_v, sem):
        wid = lax.axis_index("s") * nc + lax.axis_index("c")
        base = wid * per_w
        pltpu.sync_copy(idx_hbm.at[pl.ds(base, per_w)], idx_v)
        pltpu.async_copy(table_hbm.at[idx_v], rows_v, sem).wait()
        pltpu.sync_copy(rows_v, out_hbm.at[pl.ds(base, per_w)])

    return gather_kernel(table, idx_flat)


def _build_embt(rows_t_ref, embt_ref):
    embt_ref[0:EMB_DIM, :] = rows_t_ref[
        0:EMB_DIM, 0:BATCH_SIZE].astype(jnp.bfloat16)
    embt_ref[EMB_DIM:2 * EMB_DIM, :] = rows_t_ref[
        0:EMB_DIM, BATCH_SIZE:2 * BATCH_SIZE].astype(jnp.bfloat16)


def _logits_t(w_ref, b_ref, embt_ref):
    wb = w_ref[...].astype(jnp.bfloat16)
    zt = lax.dot_general(wb, embt_ref[...],
                         (((0,), (0,)), ((), ())),
                         preferred_element_type=jnp.float32)
    bcol = lax.dot_general(b_ref[0], jnp.ones((1, 1), jnp.float32),
                           (((0,), (0,)), ((), ())),
                           preferred_element_type=jnp.float32)
    return zt + bcol


def _pass1_body(rows_t_ref, w_ref, b_ref, m_ref, l_ref, embt_ref):
    j = pl.program_id(0)

    @pl.when(j == 0)
    def _():
        _build_embt(rows_t_ref, embt_ref)
        m_ref[...] = jnp.zeros_like(m_ref)
        l_ref[...] = jnp.zeros_like(l_ref)

    r = jnp.maximum(_logits_t(w_ref, b_ref, embt_ref), 0.0)
    row = j * VT + lax.broadcasted_iota(jnp.int32, r.shape, 0)
    r = jnp.where(row < VOCAB_SIZE, r, -jnp.inf)
    tile_max = jnp.max(r, axis=0, keepdims=True)
    m_old = m_ref[...]
    m_new = jnp.maximum(m_old, tile_max)
    e = jnp.exp(r - m_new)
    l_ref[...] = l_ref[...] * jnp.exp(m_old - m_new) + jnp.sum(
        e, axis=0, keepdims=True)
    m_ref[...] = m_new


def _pass2_body(rows_t_ref, w_ref, b_ref, m_ref, l_ref, out_ref, embt_ref):
    j = pl.program_id(0)

    @pl.when(j == 0)
    def _():
        _build_embt(rows_t_ref, embt_ref)

    r = jnp.maximum(_logits_t(w_ref, b_ref, embt_ref), 0.0)
    out_ref[...] = jnp.exp(r - m_ref[...]) * (1.0 / l_ref[...])


def kernel(inputs, E, W, b):
    table = _prep_table(E.T)
    idx = jnp.concatenate([inputs[:, 0], inputs[:, 1]]).astype(jnp.int32)
    rows = _sc_gather(table, idx)
    rows_t = rows.T
    b3 = jnp.pad(b, (0, NV * VT - VOCAB_SIZE)).reshape(NV, 1, VT)

    m, l = pl.pallas_call(
        _pass1_body,
        grid=(NV,),
        in_specs=[
            pl.BlockSpec((EMB_PAD, 2 * BATCH_SIZE), lambda j: (0, 0)),
            pl.BlockSpec((2 * EMB_DIM, VT), lambda j: (0, j)),
            pl.BlockSpec((1, 1, VT), lambda j: (j, 0, 0)),
        ],
        out_specs=[
            pl.BlockSpec((1, BATCH_SIZE), lambda j: (0, 0)),
            pl.BlockSpec((1, BATCH_SIZE), lambda j: (0, 0)),
        ],
        out_shape=[
            jax.ShapeDtypeStruct((1, BATCH_SIZE), jnp.float32),
            jax.ShapeDtypeStruct((1, BATCH_SIZE), jnp.float32),
        ],
        scratch_shapes=[pltpu.VMEM((2 * EMB_DIM, BATCH_SIZE), jnp.bfloat16)],
        compiler_params=pltpu.CompilerParams(
            dimension_semantics=("arbitrary",)),
    )(rows_t, W, b3)

    out_t = pl.pallas_call(
        _pass2_body,
        grid=(NV,),
        in_specs=[
            pl.BlockSpec((EMB_PAD, 2 * BATCH_SIZE), lambda j: (0, 0)),
            pl.BlockSpec((2 * EMB_DIM, VT), lambda j: (0, j)),
            pl.BlockSpec((1, 1, VT), lambda j: (j, 0, 0)),
            pl.BlockSpec((1, BATCH_SIZE), lambda j: (0, 0)),
            pl.BlockSpec((1, BATCH_SIZE), lambda j: (0, 0)),
        ],
        out_specs=pl.BlockSpec((VT, BATCH_SIZE), lambda j: (j, 0)),
        out_shape=jax.ShapeDtypeStruct((VOCAB_SIZE, BATCH_SIZE), jnp.float32),
        scratch_shapes=[pltpu.VMEM((2 * EMB_DIM, BATCH_SIZE), jnp.bfloat16)],
        compiler_params=pltpu.CompilerParams(
            dimension_semantics=("arbitrary",)),
    )(rows_t, W, b3, m, l)
    return out_t.T


# speedup vs baseline: 1.3644x; 1.0578x over previous
import functools

import jax
import jax.numpy as jnp
from jax import lax
from jax.experimental import pallas as pl
from jax.experimental.pallas import tpu as pltpu
from jax.experimental.pallas import tpu_sc as plsc

VOCAB_SIZE = 100000
EMB_DIM = 100
EMB_PAD = 128
BATCH_SIZE = 1024
VT = 4096
NV = (VOCAB_SIZE + VT - 1) // VT


def _padt_body(et_ref, out_ref):
    z = et_ref[...]
    z = jnp.concatenate(
        [z, jnp.zeros((EMB_PAD - EMB_DIM, VT), jnp.float32)], axis=0)
    out_ref[...] = z.T


def _prep_table(Et):
    return pl.pallas_call(
        _padt_body,
        grid=(NV,),
        in_specs=[pl.BlockSpec((EMB_DIM, VT), lambda j: (0, j))],
        out_specs=pl.BlockSpec((VT, EMB_PAD), lambda j: (j, 0)),
        out_shape=jax.ShapeDtypeStruct((VOCAB_SIZE, EMB_PAD), jnp.float32),
        compiler_params=pltpu.CompilerParams(
            dimension_semantics=("arbitrary",)),
    )(Et)


def _sc_gather(table, idx_flat):
    nc, ns = 2, 16
    nw = nc * ns
    n_idx = idx_flat.shape[0]
    per_w = n_idx // nw
    mesh = plsc.VectorSubcoreMesh(core_axis_name="c", subcore_axis_name="s",
                                  num_cores=nc, num_subcores=ns)

    @functools.partial(
        pl.kernel,
        mesh=mesh,
        out_type=jax.ShapeDtypeStruct((n_idx, EMB_PAD), jnp.float32),
        scratch_types=[
            pltpu.VMEM((per_w,), jnp.int32),
            pltpu.VMEM((per_w, EMB_PAD), jnp.float32),
            pltpu.SemaphoreType.DMA,
        ],
    )
    def gather_kernel(table_hbm, idx_hbm, out_hbm, idx_v, rows_v, sem):
        wid = lax.axis_index("s") * nc + lax.axis_index("c")
        base = wid * per_w
        pltpu.sync_copy(idx_hbm.at[pl.ds(base, per_w)], idx_v)
        pltpu.async_copy(table_hbm.at[idx_v], rows_v, sem).wait()
        pltpu.sync_copy(rows_v, out_hbm.at[pl.ds(base, per_w)])

    return gather_kernel(table, idx_flat)


def _build_embt(rows_t_ref, embt_ref):
    embt_ref[0:EMB_DIM, :] = rows_t_ref[
        0:EMB_DIM, 0:BATCH_SIZE].astype(jnp.bfloat16)
    embt_ref[EMB_DIM:2 * EMB_DIM, :] = rows_t_ref[
        0:EMB_DIM, BATCH_SIZE:2 * BATCH_SIZE].astype(jnp.bfloat16)


def _logits_t(w_ref, b_ref, embt_ref):
    wb = w_ref[...].astype(jnp.bfloat16)
    zt = lax.dot_general(wb, embt_ref[...],
                         (((0,), (0,)), ((), ())),
                         preferred_element_type=jnp.float32)
    bcol = lax.dot_general(b_ref[0], jnp.ones((1, 1), jnp.float32),
                           (((0,), (0,)), ((), ())),
                           preferred_element_type=jnp.float32)
    return zt + bcol


def _pass1_body(rows_t_ref, w_ref, b_ref, m_ref, l_ref, embt_ref):
    j = pl.program_id(0)

    @pl.when(j == 0)
    def _():
        _build_embt(rows_t_ref, embt_ref)
        m_ref[...] = jnp.zeros_like(m_ref)
        l_ref[...] = jnp.zeros_like(l_ref)

    r = jnp.maximum(_logits_t(w_ref, b_ref, embt_ref), 0.0)
    row = j * VT + lax.broadcasted_iota(jnp.int32, r.shape, 0)
    r = jnp.where(row < VOCAB_SIZE, r, -jnp.inf)
    tile_max = jnp.max(r, axis=0, keepdims=True)
    m_old = m_ref[...]
    m_new = jnp.maximum(m_old, tile_max)
    e = jnp.exp(r - m_new)
    l_ref[...] = l_ref[...] * jnp.exp(m_old - m_new) + jnp.sum(
        e, axis=0, keepdims=True)
    m_ref[...] = m_new


def _pass2_body(rows_t_ref, w_ref, b_ref, m_ref, l_ref, out_ref, embt_ref):
    j = pl.program_id(0)

    @pl.when(j == 0)
    def _():
        _build_embt(rows_t_ref, embt_ref)

    r = jnp.maximum(_logits_t(w_ref, b_ref, embt_ref), 0.0)
    out_ref[...] = jnp.exp(r - m_ref[...]) * (1.0 / l_ref[...])


def kernel(inputs, E, W, b):
    table = _prep_table(E.T)
    idx = jnp.concatenate([inputs[:, 0], inputs[:, 1]]).astype(jnp.int32)
    rows = _sc_gather(table, idx)
    rows_t = rows.T
    b3 = jnp.pad(b, (0, NV * VT - VOCAB_SIZE)).reshape(NV, 1, VT)

    m, l = pl.pallas_call(
        _pass1_body,
        grid=(NV,),
        in_specs=[
            pl.BlockSpec((EMB_PAD, 2 * BATCH_SIZE), lambda j: (0, 0)),
            pl.BlockSpec((2 * EMB_DIM, VT), lambda j: (0, j)),
            pl.BlockSpec((1, 1, VT), lambda j: (j, 0, 0)),
        ],
        out_specs=[
            pl.BlockSpec((1, BATCH_SIZE), lambda j: (0, 0)),
            pl.BlockSpec((1, BATCH_SIZE), lambda j: (0, 0)),
        ],
        out_shape=[
            jax.ShapeDtypeStruct((1, BATCH_SIZE), jnp.float32),
            jax.ShapeDtypeStruct((1, BATCH_SIZE), jnp.float32),
        ],
        scratch_shapes=[pltpu.VMEM((2 * EMB_DIM, BATCH_SIZE), jnp.bfloat16)],
        compiler_params=pltpu.CompilerParams(
            dimension_semantics=("arbitrary",)),
    )(rows_t, W, b3)

    out_t = pl.pallas_call(
        _pass2_body,
        grid=(NV,),
        in_specs=[
            pl.BlockSpec((EMB_PAD, 2 * BATCH_SIZE), lambda j: (0, 0)),
            pl.BlockSpec((2 * EMB_DIM, VT), lambda j: (0, j)),
            pl.BlockSpec((1, 1, VT), lambda j: (j, 0, 0)),
            pl.BlockSpec((1, BATCH_SIZE), lambda j: (0, 0)),
            pl.BlockSpec((1, BATCH_SIZE), lambda j: (0, 0)),
        ],
        out_specs=pl.BlockSpec((VT, BATCH_SIZE), lambda j: (j, 0)),
        out_shape=jax.ShapeDtypeStruct((VOCAB_SIZE, BATCH_SIZE), jnp.float32),
        scratch_shapes=[pltpu.VMEM((2 * EMB_DIM, BATCH_SIZE), jnp.bfloat16)],
        compiler_params=pltpu.CompilerParams(
            dimension_semantics=("arbitrary",)),
    )(rows_t, W, b3, m, l)
    return out_t.T
